# Initial kernel scaffold; baseline (speedup 1.0000x reference)
import jax
import jax.numpy as jnp
from jax.experimental import pallas as pl


def kernel(x, lengths, wih0, whh0, b0, wih1, whh1, b1, wfc, bfc):
    raise NotImplementedError("write your pallas kernel here")


# speedup vs baseline: 1.5597x; 1.5597x over previous
import functools

import jax
import jax.numpy as jnp
from jax.experimental import pallas as pl
from jax.experimental.pallas import tpu as pltpu

D_IN = 768
H = 128
HH = 2 * H
O_REAL = 85
OP = 128
NEG_BIG = -1e30
TT = 16
BT = 256


def _round_up(x, m):
    return (x + m - 1) // m * m


def _rnn_kernel(maxlen_ref, len_ref, x_ref, w1_ref, brow_ref, w2_ref,
                b1_ref, wfc_ref, bfc_ref, out_ref, h_ref, last_ref):
    b = pl.program_id(0)
    t = pl.program_id(1)
    n_t = pl.num_programs(1)
    tbase = t * TT

    @pl.when(t == 0)
    def _init():
        h_ref[...] = jnp.zeros_like(h_ref)
        last_ref[...] = jnp.zeros_like(last_ref)

    @pl.when(tbase <= maxlen_ref[b])
    def _compute():
        len_m1 = len_ref[...] - 1
        h = h_ref[...]
        last = last_ref[...]
        w2 = w2_ref[...]
        for s in range(TT):
            xs = x_ref[:, s * D_IN:(s + 1) * D_IN]
            xp = (jnp.dot(xs, w1_ref[...],
                          preferred_element_type=jnp.float32)
                  + brow_ref[...])
            p = jnp.dot(h, w2, preferred_element_type=jnp.float32)
            h = jnp.tanh(p + xp)
            g = tbase + s
            last = jnp.where(len_m1 == g - 1, h[:, H:], last)
        h_ref[...] = h
        last_ref[...] = last

    @pl.when(t == n_t - 1)
    def _final():
        len_m1 = len_ref[...] - 1
        pf = jnp.dot(h_ref[...], w2_ref[...],
                     preferred_element_type=jnp.float32)[:, H:] + b1_ref[...]
        h1f = jnp.tanh(pf)
        last = jnp.where(len_m1 == n_t * TT - 1, h1f, last_ref[...])
        logits = (jnp.dot(last, wfc_ref[...],
                          preferred_element_type=jnp.float32) + bfc_ref[...])
        m = jnp.max(logits, axis=1, keepdims=True)
        sh = logits - m
        lse = jnp.log(jnp.sum(jnp.exp(sh), axis=1, keepdims=True))
        out_ref[...] = sh - lse


@functools.partial(jax.jit, static_argnames=())
def kernel(x, lengths, wih0, whh0, b0, wih1, whh1, b1, wfc, bfc):
    B, T, D = x.shape
    B_pad = _round_up(max(B, 8), 8)
    bt = BT if B_pad % BT == 0 else B_pad
    n_b = B_pad // bt
    T_pad = _round_up(T, TT)
    n_t = T_pad // TT

    xf = x.astype(jnp.float32)
    if B_pad != B or T_pad != T:
        xf = jnp.pad(xf, ((0, B_pad - B), (0, T_pad - T), (0, 0)))
    x2 = xf.reshape(B_pad, T_pad * D)

    len_pad = lengths.astype(jnp.int32)
    if B_pad != B:
        len_pad = jnp.pad(len_pad, (0, B_pad - B), constant_values=1)
    len_col = len_pad.reshape(B_pad, 1)
    maxlen = jnp.max(len_pad.reshape(n_b, bt), axis=1).astype(jnp.int32)

    w1 = jnp.zeros((D, HH), jnp.float32).at[:, :H].set(wih0.T)
    brow = jnp.concatenate([b0, b1]).reshape(1, HH)
    w2 = jnp.zeros((HH, HH), jnp.float32)
    w2 = w2.at[:H, :H].set(whh0.T).at[:H, H:].set(wih1.T).at[H:, H:].set(whh1.T)
    b1row = b1.reshape(1, H)
    wfcp = jnp.zeros((H, OP), jnp.float32).at[:, :O_REAL].set(wfc.T)
    bfcp = jnp.full((1, OP), NEG_BIG, jnp.float32).at[0, :O_REAL].set(bfc)

    out_pad = pl.pallas_call(
        _rnn_kernel,
        out_shape=jax.ShapeDtypeStruct((B_pad, OP), jnp.float32),
        grid_spec=pltpu.PrefetchScalarGridSpec(
            num_scalar_prefetch=1,
            grid=(n_b, n_t),
            in_specs=[
                pl.BlockSpec((bt, 1), lambda b, t, ml: (b, 0)),
                pl.BlockSpec((bt, TT * D), lambda b, t, ml: (b, t)),
                pl.BlockSpec((D, HH), lambda b, t, ml: (0, 0)),
                pl.BlockSpec((1, HH), lambda b, t, ml: (0, 0)),
                pl.BlockSpec((HH, HH), lambda b, t, ml: (0, 0)),
                pl.BlockSpec((1, H), lambda b, t, ml: (0, 0)),
                pl.BlockSpec((H, OP), lambda b, t, ml: (0, 0)),
                pl.BlockSpec((1, OP), lambda b, t, ml: (0, 0)),
            ],
            out_specs=pl.BlockSpec((bt, OP), lambda b, t, ml: (b, 0)),
            scratch_shapes=[
                pltpu.VMEM((bt, HH), jnp.float32),
                pltpu.VMEM((bt, H), jnp.float32),
            ],
        ),
        compiler_params=pltpu.CompilerParams(
            dimension_semantics=("parallel", "arbitrary"),
            vmem_limit_bytes=60 * 2**20),
    )(maxlen, len_col, x2, w1, brow, w2, b1row, wfcp, bfcp)

    return out_pad[:B, :O_REAL]


# speedup vs baseline: 3.4432x; 2.2076x over previous
import functools

import jax
import jax.numpy as jnp
from jax.experimental import pallas as pl
from jax.experimental.pallas import tpu as pltpu

D_IN = 768
H = 128
HH = 2 * H
O_REAL = 85
OP = 128
NEG_BIG = -1e30
TT = 16
BT = 256


def _round_up(x, m):
    return (x + m - 1) // m * m


def _rnn_kernel(maxlen_ref, len_ref, x_ref, w1_ref, brow_ref, w2_ref,
                b1_ref, wfc_ref, bfc_ref, out_ref, xp_ref, h_ref, last_ref):
    b = pl.program_id(0)
    t = pl.program_id(1)
    n_t = pl.num_programs(1)
    tbase = t * TT

    @pl.when(t == 0)
    def _init():
        h_ref[...] = jnp.zeros_like(h_ref)
        last_ref[...] = jnp.zeros_like(last_ref)

    @pl.when(tbase <= maxlen_ref[b])
    def _compute():
        bt = h_ref.shape[0]
        xv = x_ref[...].reshape(bt * TT, D_IN)
        xp_all = (jnp.dot(xv, w1_ref[...],
                          preferred_element_type=jnp.float32)
                  + brow_ref[...])
        xp3 = xp_all.reshape(bt, TT, HH)
        for s in range(TT):
            xp_ref[s * bt:(s + 1) * bt, :] = xp3[:, s, :]

        len_m1 = len_ref[...] - 1
        h = h_ref[...]
        last = last_ref[...]
        w2 = w2_ref[...]
        for s in range(TT):
            p = jnp.dot(h, w2, preferred_element_type=jnp.float32)
            h = jnp.tanh(p + xp_ref[s * bt:(s + 1) * bt, :])
            g = tbase + s
            last = jnp.where(len_m1 == g - 1, h[:, H:], last)
        h_ref[...] = h
        last_ref[...] = last

    @pl.when(t == n_t - 1)
    def _final():
        len_m1 = len_ref[...] - 1
        pf = jnp.dot(h_ref[...], w2_ref[...],
                     preferred_element_type=jnp.float32)[:, H:] + b1_ref[...]
        h1f = jnp.tanh(pf)
        last = jnp.where(len_m1 == n_t * TT - 1, h1f, last_ref[...])
        logits = (jnp.dot(last, wfc_ref[...],
                          preferred_element_type=jnp.float32) + bfc_ref[...])
        m = jnp.max(logits, axis=1, keepdims=True)
        sh = logits - m
        lse = jnp.log(jnp.sum(jnp.exp(sh), axis=1, keepdims=True))
        out_ref[...] = sh - lse


@functools.partial(jax.jit, static_argnames=())
def kernel(x, lengths, wih0, whh0, b0, wih1, whh1, b1, wfc, bfc):
    B, T, D = x.shape
    B_pad = _round_up(max(B, 8), 8)
    bt = BT if B_pad % BT == 0 else B_pad
    n_b = B_pad // bt
    T_pad = _round_up(T, TT)
    n_t = T_pad // TT

    xf = x.astype(jnp.float32)
    if B_pad != B or T_pad != T:
        xf = jnp.pad(xf, ((0, B_pad - B), (0, T_pad - T), (0, 0)))

    len_pad = lengths.astype(jnp.int32)
    if B_pad != B:
        len_pad = jnp.pad(len_pad, (0, B_pad - B), constant_values=1)
    len_col = len_pad.reshape(B_pad, 1)
    maxlen = jnp.max(len_pad.reshape(n_b, bt), axis=1).astype(jnp.int32)

    w1 = jnp.zeros((D, HH), jnp.float32).at[:, :H].set(wih0.T)
    brow = jnp.concatenate([b0, b1]).reshape(1, HH)
    w2 = jnp.zeros((HH, HH), jnp.float32)
    w2 = w2.at[:H, :H].set(whh0.T).at[:H, H:].set(wih1.T).at[H:, H:].set(whh1.T)
    b1row = b1.reshape(1, H)
    wfcp = jnp.zeros((H, OP), jnp.float32).at[:, :O_REAL].set(wfc.T)
    bfcp = jnp.full((1, OP), NEG_BIG, jnp.float32).at[0, :O_REAL].set(bfc)

    out_pad = pl.pallas_call(
        _rnn_kernel,
        out_shape=jax.ShapeDtypeStruct((B_pad, OP), jnp.float32),
        grid_spec=pltpu.PrefetchScalarGridSpec(
            num_scalar_prefetch=1,
            grid=(n_b, n_t),
            in_specs=[
                pl.BlockSpec((bt, 1), lambda b, t, ml: (b, 0)),
                pl.BlockSpec((bt, TT, D), lambda b, t, ml: (b, t, 0)),
                pl.BlockSpec((D, HH), lambda b, t, ml: (0, 0)),
                pl.BlockSpec((1, HH), lambda b, t, ml: (0, 0)),
                pl.BlockSpec((HH, HH), lambda b, t, ml: (0, 0)),
                pl.BlockSpec((1, H), lambda b, t, ml: (0, 0)),
                pl.BlockSpec((H, OP), lambda b, t, ml: (0, 0)),
                pl.BlockSpec((1, OP), lambda b, t, ml: (0, 0)),
            ],
            out_specs=pl.BlockSpec((bt, OP), lambda b, t, ml: (b, 0)),
            scratch_shapes=[
                pltpu.VMEM((TT * bt, HH), jnp.float32),
                pltpu.VMEM((bt, HH), jnp.float32),
                pltpu.VMEM((bt, H), jnp.float32),
            ],
        ),
        compiler_params=pltpu.CompilerParams(
            dimension_semantics=("parallel", "arbitrary"),
            vmem_limit_bytes=60 * 2**20),
    )(maxlen, len_col, xf, w1, brow, w2, b1row, wfcp, bfcp)

    return out_pad[:B, :O_REAL]


# speedup vs baseline: 3.6990x; 1.0743x over previous
import functools

import jax
import jax.numpy as jnp
from jax.experimental import pallas as pl
from jax.experimental.pallas import tpu as pltpu

D_IN = 768
H = 128
HH = 2 * H
O_REAL = 85
OP = 128
NEG_BIG = -1e30
TT = 16
BT = 256


def _round_up(x, m):
    return (x + m - 1) // m * m


def _rnn_kernel(len_ref, x_ref, w1_ref, brow_ref, w2_ref,
                b1_ref, wfc_ref, bfc_ref, out_ref, xp_ref, h_ref, last_ref):
    t = pl.program_id(1)
    n_t = pl.num_programs(1)
    tbase = t * TT

    @pl.when(t == 0)
    def _init():
        h_ref[...] = jnp.zeros_like(h_ref)
        last_ref[...] = jnp.zeros_like(last_ref)

    @pl.when(tbase <= jnp.max(len_ref[...]))
    def _compute():
        bt = h_ref.shape[0]
        xv = x_ref[...].reshape(bt * TT, D_IN)
        xp_all = (jnp.dot(xv, w1_ref[...],
                          preferred_element_type=jnp.float32)
                  + brow_ref[...])
        xp3 = xp_all.reshape(bt, TT, HH)
        for s in range(TT):
            xp_ref[s * bt:(s + 1) * bt, :] = xp3[:, s, :]

        len_m1 = len_ref[...] - 1
        h = h_ref[...]
        last = last_ref[...]
        w2 = w2_ref[...]
        for s in range(TT):
            p = jnp.dot(h, w2, preferred_element_type=jnp.float32)
            h = jnp.tanh(p + xp_ref[s * bt:(s + 1) * bt, :])
            g = tbase + s
            last = jnp.where(len_m1 == g - 1, h[:, H:], last)
        h_ref[...] = h
        last_ref[...] = last

    @pl.when(t == n_t - 1)
    def _final():
        len_m1 = len_ref[...] - 1
        pf = jnp.dot(h_ref[...], w2_ref[...],
                     preferred_element_type=jnp.float32)[:, H:] + b1_ref[...]
        h1f = jnp.tanh(pf)
        last = jnp.where(len_m1 == n_t * TT - 1, h1f, last_ref[...])
        logits = (jnp.dot(last, wfc_ref[...],
                          preferred_element_type=jnp.float32) + bfc_ref[...])
        m = jnp.max(logits, axis=1, keepdims=True)
        sh = logits - m
        lse = jnp.log(jnp.sum(jnp.exp(sh), axis=1, keepdims=True))
        out_ref[...] = sh - lse


@functools.partial(jax.jit, static_argnames=())
def kernel(x, lengths, wih0, whh0, b0, wih1, whh1, b1, wfc, bfc):
    B, T, D = x.shape
    B_pad = _round_up(max(B, 8), 8)
    bt = BT if B_pad % BT == 0 else B_pad
    n_b = B_pad // bt
    T_pad = _round_up(T, TT)
    n_t = T_pad // TT

    xf = x.astype(jnp.float32)
    if B_pad != B or T_pad != T:
        xf = jnp.pad(xf, ((0, B_pad - B), (0, T_pad - T), (0, 0)))

    len_pad = lengths.astype(jnp.int32)
    if B_pad != B:
        len_pad = jnp.pad(len_pad, (0, B_pad - B), constant_values=1)
    len_col = len_pad.reshape(B_pad, 1)

    w1 = jnp.pad(wih0, ((0, H), (0, 0))).T
    brow = jnp.concatenate([b0, b1]).reshape(1, HH)
    w2 = jnp.concatenate(
        [jnp.pad(whh0, ((0, 0), (0, H))),
         jnp.concatenate([wih1, whh1], axis=1)],
        axis=0).T
    b1row = b1.reshape(1, H)
    wfcp = jnp.pad(wfc, ((0, OP - O_REAL), (0, 0))).T
    bfcp = jnp.concatenate(
        [bfc, jnp.full((OP - O_REAL,), NEG_BIG, jnp.float32)]).reshape(1, OP)

    out = pl.pallas_call(
        _rnn_kernel,
        out_shape=jax.ShapeDtypeStruct((B_pad, O_REAL), jnp.float32),
        grid=(n_b, n_t),
        in_specs=[
            pl.BlockSpec((bt, 1), lambda b, t: (b, 0)),
            pl.BlockSpec((bt, TT, D), lambda b, t: (b, t, 0)),
            pl.BlockSpec((D, HH), lambda b, t: (0, 0)),
            pl.BlockSpec((1, HH), lambda b, t: (0, 0)),
            pl.BlockSpec((HH, HH), lambda b, t: (0, 0)),
            pl.BlockSpec((1, H), lambda b, t: (0, 0)),
            pl.BlockSpec((H, OP), lambda b, t: (0, 0)),
            pl.BlockSpec((1, OP), lambda b, t: (0, 0)),
        ],
        out_specs=pl.BlockSpec((bt, OP), lambda b, t: (b, 0)),
        scratch_shapes=[
            pltpu.VMEM((TT * bt, HH), jnp.float32),
            pltpu.VMEM((bt, HH), jnp.float32),
            pltpu.VMEM((bt, H), jnp.float32),
        ],
        compiler_params=pltpu.CompilerParams(
            dimension_semantics=("parallel", "arbitrary"),
            vmem_limit_bytes=60 * 2**20),
    )(len_col, xf, w1, brow, w2, b1row, wfcp, bfcp)

    return out if B_pad == B else out[:B]
